# baseline (speedup 1.0000x reference)
import functools

import jax
import jax.numpy as jnp
from jax import lax
from jax.experimental import pallas as pl
from jax.experimental.pallas import tpu as pltpu
from jax.experimental.pallas import tpu_sc as plsc

B, S, T, M = 32, 200, 1000, 80
N_SPK, N_EMO = 256, 8

CH = 200
NCHUNK = T // CH
L = 16
NP = M // L


def _sc_mae_kernel(mel_hbm, out_hbm, post_hbm, mexp_hbm,
                   om_hbm, op_hbm,
                   mel_v, out_v, post_v, mexp_v, acc_v):
    w = lax.axis_index("s") * 2 + lax.axis_index("c")

    accm = jnp.zeros((L,), jnp.float32)
    accp = jnp.zeros((L,), jnp.float32)

    for c in range(NCHUNK):
        t0 = c * CH
        pltpu.sync_copy(mel_hbm.at[w, pl.ds(t0, CH), :], mel_v)
        pltpu.sync_copy(out_hbm.at[w, pl.ds(t0, CH), :], out_v)
        pltpu.sync_copy(post_hbm.at[w, pl.ds(t0, CH), :], post_v)
        pltpu.sync_copy(mexp_hbm.at[w, pl.ds(t0, CH), :], mexp_v)

        def body(f, carry):
            am, ap = carry
            mvec = mexp_v[f, :]
            s1 = jnp.zeros((L,), jnp.float32)
            s2 = jnp.zeros((L,), jnp.float32)
            for p in range(NP):
                melv = mel_v[f, pl.ds(p * L, L)]
                ov = out_v[f, pl.ds(p * L, L)]
                pv = post_v[f, pl.ds(p * L, L)]
                s1 = s1 + jnp.abs(ov - melv)
                s2 = s2 + jnp.abs(pv - melv)
            am = am + mvec * s1
            ap = ap + mvec * s2
            return am, ap

        accm, accp = lax.fori_loop(0, CH, body, (accm, accp))

    acc_v[...] = accm
    pltpu.sync_copy(acc_v, om_hbm.at[w])
    acc_v[...] = accp
    pltpu.sync_copy(acc_v, op_hbm.at[w])


def _sc_mae(mels, output, postnet, mask_exp):
    mesh = plsc.VectorSubcoreMesh(core_axis_name="c", subcore_axis_name="s")
    f = functools.partial(
        pl.kernel,
        mesh=mesh,
        out_type=[jax.ShapeDtypeStruct((B, L), jnp.float32),
                  jax.ShapeDtypeStruct((B, L), jnp.float32)],
        scratch_types=[
            pltpu.VMEM((CH, M), jnp.float32),
            pltpu.VMEM((CH, M), jnp.float32),
            pltpu.VMEM((CH, M), jnp.float32),
            pltpu.VMEM((CH, L), jnp.float32),
            pltpu.VMEM((L,), jnp.float32),
        ],
    )(_sc_mae_kernel)
    return f(mels, output, postnet, mask_exp)


def _ce(logits, labels):
    mx = jnp.max(logits, axis=-1, keepdims=True)
    lse = mx + jnp.log(jnp.sum(jnp.exp(logits - mx), axis=-1, keepdims=True))
    logp = logits - lse
    onehot = lax.broadcasted_iota(jnp.int32, logits.shape, 1) == labels
    picked = jnp.sum(jnp.where(onehot, logp, 0.0))
    return -picked / float(B)


def _tc_body(pm_ref, pp_ref, mv_ref, ppd_ref, ptd_ref, epd_ref, etd_ref,
             dp_ref, dt_ref, sv_ref,
             s1_ref, s2_ref, e1_ref, e2_ref, spk_ref, emo_ref,
             o_ref):
    mvd = mv_ref[...]
    s_mask = jnp.sum(mvd)
    denom3 = jnp.maximum(s_mask * float(M), 1.0)
    mel_loss = jnp.sum(pm_ref[...]) / denom3
    postnet_mel_loss = jnp.sum(pp_ref[...]) / denom3

    denom1 = jnp.maximum(s_mask, 1.0)
    pitch_loss = jnp.sum((ppd_ref[...] - ptd_ref[...]) ** 2 * mvd) / denom1
    energy_loss = jnp.sum((epd_ref[...] - etd_ref[...]) ** 2 * mvd) / denom1

    sv = sv_ref[...]
    log_dur = jnp.log(dt_ref[...] + 1.0)
    duration_loss = (jnp.sum((dp_ref[...] - log_dur) ** 2 * sv)
                     / jnp.maximum(jnp.sum(sv), 1.0))

    spk = spk_ref[...]
    emo = emo_ref[...]
    speaker_loss_1 = _ce(s1_ref[...], spk)
    speaker_loss_2 = _ce(s2_ref[...], spk)
    emotion_loss_1 = _ce(e1_ref[...], emo)
    emotion_loss_2 = _ce(e2_ref[...], emo)

    all_loss = (mel_loss + postnet_mel_loss + pitch_loss + energy_loss
                + duration_loss)
    total_loss = (all_loss + speaker_loss_1 + emotion_loss_1
                  + speaker_loss_2 + emotion_loss_2)

    vals = (mel_loss, postnet_mel_loss, pitch_loss, energy_loss,
            duration_loss, speaker_loss_1, speaker_loss_2,
            emotion_loss_1, emotion_loss_2, total_loss)
    col = lax.broadcasted_iota(jnp.int32, (8, 128), 1)
    row = lax.broadcasted_iota(jnp.int32, (8, 128), 0)
    res = jnp.zeros((8, 128), jnp.float32)
    for k, v in enumerate(vals):
        res = jnp.where((row == 0) & (col == k), v, res)
    o_ref[...] = res


@jax.jit
def _run(mels, pitches, energies, durations, speakers, emotions, output,
         postnet_output, p_preds, e_preds, d_preds, src_masks, mel_masks,
         spk_cls_1_output, spk_cls_2_output, emo_cls_1_output,
         emo_cls_2_output):
    mel_valid = (~mel_masks).astype(jnp.float32)
    src_valid = (~src_masks).astype(jnp.float32)
    mask_exp = jnp.broadcast_to(mel_valid[:, :, None], (B, T, L))

    pm, pp = _sc_mae(mels, output, postnet_output, mask_exp)

    dur_f = durations.astype(jnp.float32)
    spk = speakers.astype(jnp.int32).reshape(B, 1)
    emo = emotions.astype(jnp.int32).reshape(B, 1)

    whole = lambda r, c: pl.BlockSpec((r, c), lambda: (0, 0))

    out = pl.pallas_call(
        _tc_body,
        in_specs=[
            whole(B, L), whole(B, L),
            whole(B, T), whole(B, T), whole(B, T), whole(B, T), whole(B, T),
            whole(B, S), whole(B, S), whole(B, S),
            whole(B, N_SPK), whole(B, N_SPK),
            whole(B, N_EMO), whole(B, N_EMO),
            whole(B, 1), whole(B, 1),
        ],
        out_specs=pl.BlockSpec((8, 128), lambda: (0, 0)),
        out_shape=jax.ShapeDtypeStruct((8, 128), jnp.float32),
    )(pm, pp,
      mel_valid, p_preds, pitches, e_preds, energies,
      d_preds, dur_f, src_valid,
      spk_cls_1_output, spk_cls_2_output,
      emo_cls_1_output, emo_cls_2_output,
      spk, emo)
    return tuple(out[0, k] for k in range(10))


def kernel(mels, pitches, energies, durations, speakers, emotions, output,
           postnet_output, p_preds, e_preds, d_preds, src_masks, mel_masks,
           spk_cls_1_output, spk_cls_2_output, emo_cls_1_output,
           emo_cls_2_output):
    return _run(mels, pitches, energies, durations, speakers, emotions,
                output, postnet_output, p_preds, e_preds, d_preds,
                src_masks, mel_masks, spk_cls_1_output, spk_cls_2_output,
                emo_cls_1_output, emo_cls_2_output)


# speedup vs baseline: 1.2437x; 1.2437x over previous
import functools

import jax
import jax.numpy as jnp
from jax import lax
from jax.experimental import pallas as pl
from jax.experimental.pallas import tpu as pltpu
from jax.experimental.pallas import tpu_sc as plsc

B, S, T, M = 32, 200, 1000, 80
N_SPK, N_EMO = 256, 8

CH = 40
NCHUNK = T // CH
L = 16
NP = M // L

_GDN = lax.GatherDimensionNumbers(
    offset_dims=(), collapsed_slice_dims=(0,), start_index_map=(0,))


def _bcast_lane(vec, j):
    idx = jnp.full((L, 1), j, jnp.int32)
    return lax.gather(vec, idx, _GDN, (1,),
                      mode=lax.GatherScatterMode.PROMISE_IN_BOUNDS)


def _sc_mae_kernel(mel_hbm, out_hbm, post_hbm, mv_hbm,
                   om_hbm, op_hbm,
                   mel_v0, out_v0, post_v0, mel_v1, out_v1, post_v1,
                   mask_v, acc_v, sem0, sem1):
    w = lax.axis_index("s") * 2 + lax.axis_index("c")

    pltpu.sync_copy(mv_hbm.at[w, 0, :], mask_v)

    bufs = ((mel_v0, out_v0, post_v0, sem0),
            (mel_v1, out_v1, post_v1, sem1))

    def start(c):
        mel_v, out_v, post_v, sem = bufs[c % 2]
        t0 = c * CH
        return (
            pltpu.async_copy(mel_hbm.at[w, pl.ds(t0, CH), :], mel_v, sem),
            pltpu.async_copy(out_hbm.at[w, pl.ds(t0, CH), :], out_v, sem),
            pltpu.async_copy(post_hbm.at[w, pl.ds(t0, CH), :], post_v, sem),
        )

    accm = jnp.zeros((L,), jnp.float32)
    accp = jnp.zeros((L,), jnp.float32)

    handles = start(0)
    for c in range(NCHUNK):
        nxt = start(c + 1) if c + 1 < NCHUNK else ()
        for h in handles:
            h.wait()
        handles = nxt

        mel_v, out_v, post_v, _ = bufs[c % 2]
        t0 = c * CH

        def body(f, carry):
            am, ap = carry
            o16 = pl.multiple_of(((t0 + f) // L) * L, L)
            mrow = mask_v[pl.ds(o16, L)]
            mvec = _bcast_lane(mrow, t0 + f - o16)
            s1 = None
            s2 = None
            for p in range(NP):
                melv = mel_v[f, pl.ds(p * L, L)]
                ov = out_v[f, pl.ds(p * L, L)]
                pv = post_v[f, pl.ds(p * L, L)]
                a1 = jnp.abs(ov - melv)
                a2 = jnp.abs(pv - melv)
                s1 = a1 if s1 is None else s1 + a1
                s2 = a2 if s2 is None else s2 + a2
            am = am + mvec * s1
            ap = ap + mvec * s2
            return am, ap

        accm, accp = lax.fori_loop(0, CH, body, (accm, accp))

    acc_v[...] = accm
    pltpu.sync_copy(acc_v, om_hbm.at[w])
    acc_v[...] = accp
    pltpu.sync_copy(acc_v, op_hbm.at[w])


def _sc_mae(mels, output, postnet, mel_valid):
    mesh = plsc.VectorSubcoreMesh(core_axis_name="c", subcore_axis_name="s")
    f = functools.partial(
        pl.kernel,
        mesh=mesh,
        out_type=[jax.ShapeDtypeStruct((B, L), jnp.float32),
                  jax.ShapeDtypeStruct((B, L), jnp.float32)],
        scratch_types=[
            pltpu.VMEM((CH, M), jnp.float32),
            pltpu.VMEM((CH, M), jnp.float32),
            pltpu.VMEM((CH, M), jnp.float32),
            pltpu.VMEM((CH, M), jnp.float32),
            pltpu.VMEM((CH, M), jnp.float32),
            pltpu.VMEM((CH, M), jnp.float32),
            pltpu.VMEM((1024,), jnp.float32),
            pltpu.VMEM((L,), jnp.float32),
            pltpu.SemaphoreType.DMA,
            pltpu.SemaphoreType.DMA,
        ],
    )(_sc_mae_kernel)
    return f(mels, output, postnet, mel_valid)


def _ce(logits, labels):
    mx = jnp.max(logits, axis=-1, keepdims=True)
    lse = mx + jnp.log(jnp.sum(jnp.exp(logits - mx), axis=-1, keepdims=True))
    logp = logits - lse
    onehot = lax.broadcasted_iota(jnp.int32, logits.shape, 1) == labels
    picked = jnp.sum(jnp.where(onehot, logp, 0.0))
    return -picked / float(B)


def _tc_body(pm_ref, pp_ref, mv_ref, ppd_ref, ptd_ref, epd_ref, etd_ref,
             dp_ref, dt_ref, sv_ref,
             s1_ref, s2_ref, e1_ref, e2_ref, spk_ref, emo_ref,
             o_ref):
    mvd = mv_ref[...]
    s_mask = jnp.sum(mvd)
    denom3 = jnp.maximum(s_mask * float(M), 1.0)
    mel_loss = jnp.sum(pm_ref[...]) / denom3
    postnet_mel_loss = jnp.sum(pp_ref[...]) / denom3

    denom1 = jnp.maximum(s_mask, 1.0)
    pitch_loss = jnp.sum((ppd_ref[...] - ptd_ref[...]) ** 2 * mvd) / denom1
    energy_loss = jnp.sum((epd_ref[...] - etd_ref[...]) ** 2 * mvd) / denom1

    sv = sv_ref[...]
    log_dur = jnp.log(dt_ref[...] + 1.0)
    duration_loss = (jnp.sum((dp_ref[...] - log_dur) ** 2 * sv)
                     / jnp.maximum(jnp.sum(sv), 1.0))

    spk = spk_ref[...]
    emo = emo_ref[...]
    speaker_loss_1 = _ce(s1_ref[...], spk)
    speaker_loss_2 = _ce(s2_ref[...], spk)
    emotion_loss_1 = _ce(e1_ref[...], emo)
    emotion_loss_2 = _ce(e2_ref[...], emo)

    all_loss = (mel_loss + postnet_mel_loss + pitch_loss + energy_loss
                + duration_loss)
    total_loss = (all_loss + speaker_loss_1 + emotion_loss_1
                  + speaker_loss_2 + emotion_loss_2)

    vals = (mel_loss, postnet_mel_loss, pitch_loss, energy_loss,
            duration_loss, speaker_loss_1, speaker_loss_2,
            emotion_loss_1, emotion_loss_2, total_loss)
    col = lax.broadcasted_iota(jnp.int32, (8, 128), 1)
    row = lax.broadcasted_iota(jnp.int32, (8, 128), 0)
    res = jnp.zeros((8, 128), jnp.float32)
    for k, v in enumerate(vals):
        res = jnp.where((row == 0) & (col == k), v, res)
    o_ref[...] = res


@jax.jit
def _run(mels, pitches, energies, durations, speakers, emotions, output,
         postnet_output, p_preds, e_preds, d_preds, src_masks, mel_masks,
         spk_cls_1_output, spk_cls_2_output, emo_cls_1_output,
         emo_cls_2_output):
    mel_valid = (~mel_masks).astype(jnp.float32)
    src_valid = (~src_masks).astype(jnp.float32)

    mv_pad = jnp.pad(mel_valid, ((0, 0), (0, 1024 - T))).reshape(B, 1, 1024)
    pm, pp = _sc_mae(mels, output, postnet_output, mv_pad)

    dur_f = durations.astype(jnp.float32)
    spk = speakers.astype(jnp.int32).reshape(B, 1)
    emo = emotions.astype(jnp.int32).reshape(B, 1)

    whole = lambda r, c: pl.BlockSpec((r, c), lambda: (0, 0))

    out = pl.pallas_call(
        _tc_body,
        in_specs=[
            whole(B, L), whole(B, L),
            whole(B, T), whole(B, T), whole(B, T), whole(B, T), whole(B, T),
            whole(B, S), whole(B, S), whole(B, S),
            whole(B, N_SPK), whole(B, N_SPK),
            whole(B, N_EMO), whole(B, N_EMO),
            whole(B, 1), whole(B, 1),
        ],
        out_specs=pl.BlockSpec((8, 128), lambda: (0, 0)),
        out_shape=jax.ShapeDtypeStruct((8, 128), jnp.float32),
    )(pm, pp,
      mel_valid, p_preds, pitches, e_preds, energies,
      d_preds, dur_f, src_valid,
      spk_cls_1_output, spk_cls_2_output,
      emo_cls_1_output, emo_cls_2_output,
      spk, emo)
    return tuple(out[0, k] for k in range(10))


def kernel(mels, pitches, energies, durations, speakers, emotions, output,
           postnet_output, p_preds, e_preds, d_preds, src_masks, mel_masks,
           spk_cls_1_output, spk_cls_2_output, emo_cls_1_output,
           emo_cls_2_output):
    return _run(mels, pitches, energies, durations, speakers, emotions,
                output, postnet_output, p_preds, e_preds, d_preds,
                src_masks, mel_masks, spk_cls_1_output, spk_cls_2_output,
                emo_cls_1_output, emo_cls_2_output)


# speedup vs baseline: 1.2623x; 1.0149x over previous
import functools

import jax
import jax.numpy as jnp
from jax import lax
from jax.experimental import pallas as pl
from jax.experimental.pallas import tpu as pltpu
from jax.experimental.pallas import tpu_sc as plsc

B, S, T, M = 32, 200, 1000, 80
N_SPK, N_EMO = 256, 8

CH = 40
NCHUNK = T // CH
L = 16
NP = M // L

_GDN = lax.GatherDimensionNumbers(
    offset_dims=(), collapsed_slice_dims=(0,), start_index_map=(0,))


def _bcast_lane(vec, j):
    idx = jnp.full((L, 1), j, jnp.int32)
    return lax.gather(vec, idx, _GDN, (1,),
                      mode=lax.GatherScatterMode.PROMISE_IN_BOUNDS)


def _sc_mae_kernel(mel_hbm, out_hbm, post_hbm, mv_hbm,
                   om_hbm, op_hbm,
                   mel_v0, out_v0, post_v0, mel_v1, out_v1, post_v1,
                   mask_v, acc_v, sem0, sem1):
    w = lax.axis_index("s") * 2 + lax.axis_index("c")

    pltpu.sync_copy(mv_hbm.at[w, 0, :], mask_v.at[pl.ds(0, T)])

    bufs = ((mel_v0, out_v0, post_v0, sem0),
            (mel_v1, out_v1, post_v1, sem1))

    def start(c):
        mel_v, out_v, post_v, sem = bufs[c % 2]
        t0 = c * CH
        return (
            pltpu.async_copy(mel_hbm.at[w, pl.ds(t0, CH), :], mel_v, sem),
            pltpu.async_copy(out_hbm.at[w, pl.ds(t0, CH), :], out_v, sem),
            pltpu.async_copy(post_hbm.at[w, pl.ds(t0, CH), :], post_v, sem),
        )

    accm = jnp.zeros((L,), jnp.float32)
    accp = jnp.zeros((L,), jnp.float32)

    handles = start(0)
    for c in range(NCHUNK):
        nxt = start(c + 1) if c + 1 < NCHUNK else ()
        for h in handles:
            h.wait()
        handles = nxt

        mel_v, out_v, post_v, _ = bufs[c % 2]
        t0 = c * CH

        def body(f, carry):
            am, ap = carry
            o16 = pl.multiple_of(((t0 + f) // L) * L, L)
            mrow = mask_v[pl.ds(o16, L)]
            mvec = _bcast_lane(mrow, t0 + f - o16)
            s1 = None
            s2 = None
            for p in range(NP):
                melv = mel_v[f, pl.ds(p * L, L)]
                ov = out_v[f, pl.ds(p * L, L)]
                pv = post_v[f, pl.ds(p * L, L)]
                a1 = jnp.abs(ov - melv)
                a2 = jnp.abs(pv - melv)
                s1 = a1 if s1 is None else s1 + a1
                s2 = a2 if s2 is None else s2 + a2
            am = am + mvec * s1
            ap = ap + mvec * s2
            return am, ap

        accm, accp = lax.fori_loop(0, CH, body, (accm, accp))

    acc_v[...] = accm
    pltpu.sync_copy(acc_v, om_hbm.at[w])
    acc_v[...] = accp
    pltpu.sync_copy(acc_v, op_hbm.at[w])


def _sc_mae(mels, output, postnet, mel_valid):
    mesh = plsc.VectorSubcoreMesh(core_axis_name="c", subcore_axis_name="s")
    f = functools.partial(
        pl.kernel,
        mesh=mesh,
        out_type=[jax.ShapeDtypeStruct((B, L), jnp.float32),
                  jax.ShapeDtypeStruct((B, L), jnp.float32)],
        scratch_types=[
            pltpu.VMEM((CH, M), jnp.float32),
            pltpu.VMEM((CH, M), jnp.float32),
            pltpu.VMEM((CH, M), jnp.float32),
            pltpu.VMEM((CH, M), jnp.float32),
            pltpu.VMEM((CH, M), jnp.float32),
            pltpu.VMEM((CH, M), jnp.float32),
            pltpu.VMEM((1024,), jnp.float32),
            pltpu.VMEM((L,), jnp.float32),
            pltpu.SemaphoreType.DMA,
            pltpu.SemaphoreType.DMA,
        ],
    )(_sc_mae_kernel)
    return f(mels, output, postnet, mel_valid)


def _ce(logits, labels):
    mx = jnp.max(logits, axis=-1, keepdims=True)
    lse = mx + jnp.log(jnp.sum(jnp.exp(logits - mx), axis=-1, keepdims=True))
    logp = logits - lse
    onehot = lax.broadcasted_iota(jnp.int32, logits.shape, 1) == labels
    picked = jnp.sum(jnp.where(onehot, logp, 0.0))
    return -picked / float(B)


def _tc_body(pm_ref, pp_ref, mv_ref, ppd_ref, ptd_ref, epd_ref, etd_ref,
             dp_ref, dt_ref, sv_ref,
             s1_ref, s2_ref, e1_ref, e2_ref, spk_ref, emo_ref,
             o_ref):
    mvd = mv_ref[...]
    s_mask = jnp.sum(mvd)
    denom3 = jnp.maximum(s_mask * float(M), 1.0)
    mel_loss = jnp.sum(pm_ref[...]) / denom3
    postnet_mel_loss = jnp.sum(pp_ref[...]) / denom3

    denom1 = jnp.maximum(s_mask, 1.0)
    pitch_loss = jnp.sum((ppd_ref[...] - ptd_ref[...]) ** 2 * mvd) / denom1
    energy_loss = jnp.sum((epd_ref[...] - etd_ref[...]) ** 2 * mvd) / denom1

    sv = sv_ref[...]
    log_dur = jnp.log(dt_ref[...] + 1.0)
    duration_loss = (jnp.sum((dp_ref[...] - log_dur) ** 2 * sv)
                     / jnp.maximum(jnp.sum(sv), 1.0))

    spk = spk_ref[...]
    emo = emo_ref[...]
    speaker_loss_1 = _ce(s1_ref[...], spk)
    speaker_loss_2 = _ce(s2_ref[...], spk)
    emotion_loss_1 = _ce(e1_ref[...], emo)
    emotion_loss_2 = _ce(e2_ref[...], emo)

    all_loss = (mel_loss + postnet_mel_loss + pitch_loss + energy_loss
                + duration_loss)
    total_loss = (all_loss + speaker_loss_1 + emotion_loss_1
                  + speaker_loss_2 + emotion_loss_2)

    vals = (mel_loss, postnet_mel_loss, pitch_loss, energy_loss,
            duration_loss, speaker_loss_1, speaker_loss_2,
            emotion_loss_1, emotion_loss_2, total_loss)
    col = lax.broadcasted_iota(jnp.int32, (8, 128), 1)
    row = lax.broadcasted_iota(jnp.int32, (8, 128), 0)
    res = jnp.zeros((8, 128), jnp.float32)
    for k, v in enumerate(vals):
        res = jnp.where((row == 0) & (col == k), v, res)
    o_ref[...] = res


@jax.jit
def _run(mels, pitches, energies, durations, speakers, emotions, output,
         postnet_output, p_preds, e_preds, d_preds, src_masks, mel_masks,
         spk_cls_1_output, spk_cls_2_output, emo_cls_1_output,
         emo_cls_2_output):
    mel_valid = (~mel_masks).astype(jnp.float32)
    src_valid = (~src_masks).astype(jnp.float32)

    pm, pp = _sc_mae(mels, output, postnet_output, mel_valid.reshape(B, 1, T))

    dur_f = durations.astype(jnp.float32)
    spk = speakers.astype(jnp.int32).reshape(B, 1)
    emo = emotions.astype(jnp.int32).reshape(B, 1)

    whole = lambda r, c: pl.BlockSpec((r, c), lambda: (0, 0))

    out = pl.pallas_call(
        _tc_body,
        in_specs=[
            whole(B, L), whole(B, L),
            whole(B, T), whole(B, T), whole(B, T), whole(B, T), whole(B, T),
            whole(B, S), whole(B, S), whole(B, S),
            whole(B, N_SPK), whole(B, N_SPK),
            whole(B, N_EMO), whole(B, N_EMO),
            whole(B, 1), whole(B, 1),
        ],
        out_specs=pl.BlockSpec((8, 128), lambda: (0, 0)),
        out_shape=jax.ShapeDtypeStruct((8, 128), jnp.float32),
    )(pm, pp,
      mel_valid, p_preds, pitches, e_preds, energies,
      d_preds, dur_f, src_valid,
      spk_cls_1_output, spk_cls_2_output,
      emo_cls_1_output, emo_cls_2_output,
      spk, emo)
    return tuple(out[0, k] for k in range(10))


def kernel(mels, pitches, energies, durations, speakers, emotions, output,
           postnet_output, p_preds, e_preds, d_preds, src_masks, mel_masks,
           spk_cls_1_output, spk_cls_2_output, emo_cls_1_output,
           emo_cls_2_output):
    return _run(mels, pitches, energies, durations, speakers, emotions,
                output, postnet_output, p_preds, e_preds, d_preds,
                src_masks, mel_masks, spk_cls_1_output, spk_cls_2_output,
                emo_cls_1_output, emo_cls_2_output)


# speedup vs baseline: 1.2662x; 1.0031x over previous
import functools

import jax
import jax.numpy as jnp
from jax import lax
from jax.experimental import pallas as pl
from jax.experimental.pallas import tpu as pltpu
from jax.experimental.pallas import tpu_sc as plsc

B, S, T, M = 32, 200, 1000, 80
N_SPK, N_EMO = 256, 8

CH = 40
NCHUNK = T // CH
L = 16
NP = M // L

_GDN = lax.GatherDimensionNumbers(
    offset_dims=(), collapsed_slice_dims=(0,), start_index_map=(0,))


def _bcast_lane(vec, j):
    idx = jnp.full((L, 1), j, jnp.int32)
    return lax.gather(vec, idx, _GDN, (1,),
                      mode=lax.GatherScatterMode.PROMISE_IN_BOUNDS)


def _sc_mae_kernel(mel_hbm, out_hbm, post_hbm, mv_hbm,
                   om_hbm, op_hbm,
                   mel_v0, out_v0, post_v0, mel_v1, out_v1, post_v1,
                   mask_v, acc_v, sem0, sem1):
    w = lax.axis_index("s") * 2 + lax.axis_index("c")

    pltpu.sync_copy(mv_hbm.at[w, 0, :], mask_v.at[pl.ds(0, T)])

    bufs = ((mel_v0, out_v0, post_v0, sem0),
            (mel_v1, out_v1, post_v1, sem1))

    def start(c):
        mel_v, out_v, post_v, sem = bufs[c % 2]
        t0 = c * CH
        return (
            pltpu.async_copy(mel_hbm.at[w, pl.ds(t0, CH), :], mel_v, sem),
            pltpu.async_copy(out_hbm.at[w, pl.ds(t0, CH), :], out_v, sem),
            pltpu.async_copy(post_hbm.at[w, pl.ds(t0, CH), :], post_v, sem),
        )

    accm = jnp.zeros((L,), jnp.float32)
    accp = jnp.zeros((L,), jnp.float32)

    handles = start(0)
    for c in range(NCHUNK):
        nxt = start(c + 1) if c + 1 < NCHUNK else ()
        for h in handles:
            h.wait()
        handles = nxt

        mel_v, out_v, post_v, _ = bufs[c % 2]
        t0 = c * CH

        def body(f, carry):
            am, ap = carry
            o16 = pl.multiple_of(((t0 + f) // L) * L, L)
            mrow = mask_v[pl.ds(o16, L)]
            mvec = _bcast_lane(mrow, t0 + f - o16)
            s1 = None
            s2 = None
            for p in range(NP):
                melv = mel_v[f, pl.ds(p * L, L)]
                ov = out_v[f, pl.ds(p * L, L)]
                pv = post_v[f, pl.ds(p * L, L)]
                a1 = jnp.abs(ov - melv)
                a2 = jnp.abs(pv - melv)
                s1 = a1 if s1 is None else s1 + a1
                s2 = a2 if s2 is None else s2 + a2
            am = am + mvec * s1
            ap = ap + mvec * s2
            return am, ap

        accm, accp = lax.fori_loop(0, CH, body, (accm, accp))

    acc_v[...] = accm
    pltpu.sync_copy(acc_v, om_hbm.at[w])
    acc_v[...] = accp
    pltpu.sync_copy(acc_v, op_hbm.at[w])


def _sc_mae(mels, output, postnet, mel_valid):
    mesh = plsc.VectorSubcoreMesh(core_axis_name="c", subcore_axis_name="s")
    f = functools.partial(
        pl.kernel,
        mesh=mesh,
        compiler_params=pltpu.CompilerParams(use_tc_tiling_on_sc=True),
        out_type=[jax.ShapeDtypeStruct((B, L), jnp.float32),
                  jax.ShapeDtypeStruct((B, L), jnp.float32)],
        scratch_types=[
            pltpu.VMEM((CH, M), jnp.float32),
            pltpu.VMEM((CH, M), jnp.float32),
            pltpu.VMEM((CH, M), jnp.float32),
            pltpu.VMEM((CH, M), jnp.float32),
            pltpu.VMEM((CH, M), jnp.float32),
            pltpu.VMEM((CH, M), jnp.float32),
            pltpu.VMEM((1024,), jnp.float32),
            pltpu.VMEM((L,), jnp.float32),
            pltpu.SemaphoreType.DMA,
            pltpu.SemaphoreType.DMA,
        ],
    )(_sc_mae_kernel)
    return f(mels, output, postnet, mel_valid)


def _ce(logits, labels):
    mx = jnp.max(logits, axis=-1, keepdims=True)
    lse = mx + jnp.log(jnp.sum(jnp.exp(logits - mx), axis=-1, keepdims=True))
    logp = logits - lse
    onehot = lax.broadcasted_iota(jnp.int32, logits.shape, 1) == labels
    picked = jnp.sum(jnp.where(onehot, logp, 0.0))
    return -picked / float(B)


def _tc_body(pm_ref, pp_ref, mv_ref, ppd_ref, ptd_ref, epd_ref, etd_ref,
             dp_ref, dt_ref, sv_ref,
             s1_ref, s2_ref, e1_ref, e2_ref, spk_ref, emo_ref,
             o_ref):
    mvd = mv_ref[...]
    s_mask = jnp.sum(mvd)
    denom3 = jnp.maximum(s_mask * float(M), 1.0)
    mel_loss = jnp.sum(pm_ref[...]) / denom3
    postnet_mel_loss = jnp.sum(pp_ref[...]) / denom3

    denom1 = jnp.maximum(s_mask, 1.0)
    pitch_loss = jnp.sum((ppd_ref[...] - ptd_ref[...]) ** 2 * mvd) / denom1
    energy_loss = jnp.sum((epd_ref[...] - etd_ref[...]) ** 2 * mvd) / denom1

    sv = sv_ref[...]
    log_dur = jnp.log(dt_ref[...] + 1.0)
    duration_loss = (jnp.sum((dp_ref[...] - log_dur) ** 2 * sv)
                     / jnp.maximum(jnp.sum(sv), 1.0))

    spk = spk_ref[...]
    emo = emo_ref[...]
    speaker_loss_1 = _ce(s1_ref[...], spk)
    speaker_loss_2 = _ce(s2_ref[...], spk)
    emotion_loss_1 = _ce(e1_ref[...], emo)
    emotion_loss_2 = _ce(e2_ref[...], emo)

    all_loss = (mel_loss + postnet_mel_loss + pitch_loss + energy_loss
                + duration_loss)
    total_loss = (all_loss + speaker_loss_1 + emotion_loss_1
                  + speaker_loss_2 + emotion_loss_2)

    vals = (mel_loss, postnet_mel_loss, pitch_loss, energy_loss,
            duration_loss, speaker_loss_1, speaker_loss_2,
            emotion_loss_1, emotion_loss_2, total_loss)
    col = lax.broadcasted_iota(jnp.int32, (8, 128), 1)
    row = lax.broadcasted_iota(jnp.int32, (8, 128), 0)
    res = jnp.zeros((8, 128), jnp.float32)
    for k, v in enumerate(vals):
        res = jnp.where((row == 0) & (col == k), v, res)
    o_ref[...] = res


@jax.jit
def _run(mels, pitches, energies, durations, speakers, emotions, output,
         postnet_output, p_preds, e_preds, d_preds, src_masks, mel_masks,
         spk_cls_1_output, spk_cls_2_output, emo_cls_1_output,
         emo_cls_2_output):
    mel_valid = (~mel_masks).astype(jnp.float32)
    src_valid = (~src_masks).astype(jnp.float32)

    pm, pp = _sc_mae(mels, output, postnet_output, mel_valid.reshape(B, 1, T))

    dur_f = durations.astype(jnp.float32)
    spk = speakers.astype(jnp.int32).reshape(B, 1)
    emo = emotions.astype(jnp.int32).reshape(B, 1)

    whole = lambda r, c: pl.BlockSpec((r, c), lambda: (0, 0))

    out = pl.pallas_call(
        _tc_body,
        in_specs=[
            whole(B, L), whole(B, L),
            whole(B, T), whole(B, T), whole(B, T), whole(B, T), whole(B, T),
            whole(B, S), whole(B, S), whole(B, S),
            whole(B, N_SPK), whole(B, N_SPK),
            whole(B, N_EMO), whole(B, N_EMO),
            whole(B, 1), whole(B, 1),
        ],
        out_specs=pl.BlockSpec((8, 128), lambda: (0, 0)),
        out_shape=jax.ShapeDtypeStruct((8, 128), jnp.float32),
    )(pm, pp,
      mel_valid, p_preds, pitches, e_preds, energies,
      d_preds, dur_f, src_valid,
      spk_cls_1_output, spk_cls_2_output,
      emo_cls_1_output, emo_cls_2_output,
      spk, emo)
    return tuple(out[0, k] for k in range(10))


def kernel(mels, pitches, energies, durations, speakers, emotions, output,
           postnet_output, p_preds, e_preds, d_preds, src_masks, mel_masks,
           spk_cls_1_output, spk_cls_2_output, emo_cls_1_output,
           emo_cls_2_output):
    return _run(mels, pitches, energies, durations, speakers, emotions,
                output, postnet_output, p_preds, e_preds, d_preds,
                src_masks, mel_masks, spk_cls_1_output, spk_cls_2_output,
                emo_cls_1_output, emo_cls_2_output)
